# baseline (speedup 1.0000x reference)
import functools

import jax
import jax.numpy as jnp
import numpy as np
from jax import lax
from jax.experimental import pallas as pl
from jax.experimental.pallas import tpu as pltpu
from jax.experimental.pallas import tpu_sc as plsc

_NUM_CORES = 2
_NUM_SUBCORES = 16
_NUM_WORKERS = _NUM_CORES * _NUM_SUBCORES
_CHUNK = 80
_HEAD_BLOCK = 1280


def _tables_body(x_ref, wa_ref, wb_ref, b1_ref, a_ref, b_ref):
    xv = x_ref[...]
    a_ref[...] = (
        jnp.dot(xv, wa_ref[...], preferred_element_type=jnp.float32) + b1_ref[...]
    )
    b_ref[...] = jnp.dot(xv, wb_ref[...], preferred_element_type=jnp.float32)


def _make_tables(x, w1a, w1b, b1):
    n, _ = x.shape
    h = w1a.shape[1]
    return pl.pallas_call(
        _tables_body,
        out_shape=[
            jax.ShapeDtypeStruct((n, h), jnp.float32),
            jax.ShapeDtypeStruct((n, h), jnp.float32),
        ],
    )(x, w1a, w1b, b1.reshape(1, h))


def _gather_add(start, end, tab_a, tab_b):
    e = start.shape[0]
    h = tab_a.shape[1]
    per_worker = e // _NUM_WORKERS
    chunks = per_worker // _CHUNK
    pairs = chunks // 2
    mesh = plsc.VectorSubcoreMesh(core_axis_name="c", subcore_axis_name="s")

    chunks_per_half = _HEAD_BLOCK // _CHUNK

    start2 = start.reshape(e // _CHUNK, _CHUNK)
    end2 = end.reshape(e // _CHUNK, _CHUNK)

    @functools.partial(
        pl.kernel,
        mesh=mesh,
        out_type=jax.ShapeDtypeStruct((e // 2, 2 * h), jnp.float32),
        compiler_params=pltpu.CompilerParams(use_tc_tiling_on_sc=False),
        scratch_types=[
            pltpu.VMEM((chunks, _CHUNK), jnp.int32),
            pltpu.VMEM((chunks, _CHUNK), jnp.int32),
            pltpu.VMEM((_CHUNK, h), jnp.float32),
            pltpu.VMEM((_CHUNK, h), jnp.float32),
            pltpu.VMEM((_CHUNK, h), jnp.float32),
            pltpu.VMEM((_CHUNK, h), jnp.float32),
            pltpu.VMEM((_CHUNK, h), jnp.float32),
            pltpu.VMEM((_CHUNK, h), jnp.float32),
            pltpu.SemaphoreType.DMA,
            pltpu.SemaphoreType.DMA,
            pltpu.SemaphoreType.DMA,
            pltpu.SemaphoreType.DMA,
        ],
    )
    def sc_kernel(start_hbm, end_hbm, a_hbm, b_hbm, g_hbm,
                  idx_s, idx_e, a0, b0, a1, b1, o0, o1, sg0, sg1, st0, st1):
        wid = lax.axis_index("s") * _NUM_CORES + lax.axis_index("c")
        cbase = wid * chunks

        def fire(c, ba, bb, sem):
            pltpu.async_copy(a_hbm.at[idx_s.at[c]], ba, sem)
            pltpu.async_copy(b_hbm.at[idx_e.at[c]], bb, sem)

        def wait_g(ba, bb, sem):
            pltpu.make_async_copy(a_hbm.at[idx_s.at[0]], ba, sem).wait()
            pltpu.make_async_copy(b_hbm.at[idx_e.at[0]], bb, sem).wait()

        def add_rows(ba, bb, oo):
            def row_body(p, c2):
                for t in range(h // 16):
                    sl = pl.ds(t * 16, 16)
                    oo[p, sl] = ba[p, sl] + bb[p, sl]
                return c2

            lax.fori_loop(0, _CHUNK, row_body, 0, unroll=8)

        def out_slice(c):
            gc = cbase + c
            blk = lax.shift_right_logical(gc, 5)
            half = lax.bitwise_and(lax.shift_right_logical(gc, 4), 1)
            j0 = lax.bitwise_and(gc, chunks_per_half - 1) * _CHUNK
            rowbase = pl.multiple_of(blk * _HEAD_BLOCK + j0, 8)
            colbase = pl.multiple_of(half * h, 8)
            return g_hbm.at[pl.ds(rowbase, _CHUNK), pl.ds(colbase, h)]

        def wait_st(oo, sem):
            pltpu.make_async_copy(oo, out_slice(0), sem).wait()

        pltpu.sync_copy(start_hbm.at[pl.ds(cbase, chunks)], idx_s)
        pltpu.sync_copy(end_hbm.at[pl.ds(cbase, chunks)], idx_e)
        fire(0, a0, b0, sg0)

        def pair_body(j, carry):
            c = 2 * j

            @pl.when(j > 0)
            def _():
                wait_st(o0, st0)
                wait_st(o1, st1)

            fire(c + 1, a1, b1, sg1)
            wait_g(a0, b0, sg0)
            add_rows(a0, b0, o0)
            pltpu.async_copy(o0, out_slice(c), st0)
            fire(c + 2, a0, b0, sg0)
            wait_g(a1, b1, sg1)
            add_rows(a1, b1, o1)
            pltpu.async_copy(o1, out_slice(c + 1), st1)
            return carry

        lax.fori_loop(0, pairs, pair_body, 0)

        wait_g(a0, b0, sg0)
        wait_st(o0, st0)
        add_rows(a0, b0, o0)
        pltpu.sync_copy(o0, out_slice(chunks - 1))
        wait_st(o1, st1)

    return sc_kernel(start2, end2, tab_a, tab_b)


def _ln_tanh_stack(s, gain, bias):
    mu = jnp.mean(s, axis=1, keepdims=True)
    d = s - jnp.broadcast_to(mu, s.shape)
    var = jnp.mean(d * d, axis=1, keepdims=True)
    r = jnp.broadcast_to(jax.lax.rsqrt(var + 1e-5), s.shape)
    return jnp.tanh(d * r * gain[None] + bias[None])


def _head_body(g_ref, g1_ref, be1_ref, w2_ref, b2_ref, g2_ref, be2_ref, w3_ref, b3_ref, out_ref):
    hh, block = g_ref.shape[1], g_ref.shape[0]
    h = hh // 2
    v = jnp.transpose(g_ref[...])
    s = _ln_tanh_stack(v.reshape(2, h, block), g1_ref[...], be1_ref[...])

    def half_dot(w, x):
        return lax.dot_general(
            w, x, (((0,), (0,)), ((), ())), preferred_element_type=jnp.float32
        )

    w2 = w2_ref[...]
    s = jnp.stack([half_dot(w2, s[0]), half_dot(w2, s[1])]) + b2_ref[...][None]
    s = _ln_tanh_stack(s, g2_ref[...], be2_ref[...])
    w3 = w3_ref[...]
    res = jnp.concatenate([half_dot(w3, s[0]), half_dot(w3, s[1])]) + b3_ref[...]
    out_ref[...] = res[None]


def _edge_head(g2, g1, be1, w2, b2, g2n, be2, w3, b3, block):
    rows, hh = g2.shape
    h = hh // 2
    grid = rows // block
    full = lambda i: (0, 0)

    out = pl.pallas_call(
        _head_body,
        grid=(grid,),
        in_specs=[
            pl.BlockSpec((block, hh), lambda i: (i, 0)),
            pl.BlockSpec((h, 1), full),
            pl.BlockSpec((h, 1), full),
            pl.BlockSpec((h, h), full),
            pl.BlockSpec((h, 1), full),
            pl.BlockSpec((h, 1), full),
            pl.BlockSpec((h, 1), full),
            pl.BlockSpec((h, 1), full),
            pl.BlockSpec((1, 1), full),
        ],
        out_specs=pl.BlockSpec((1, 2, block), lambda i: (i, 0, 0)),
        out_shape=jax.ShapeDtypeStruct((grid, 2, block), jnp.float32),
        compiler_params=pltpu.CompilerParams(
            dimension_semantics=("parallel",),
        ),
    )(
        g2,
        g1.reshape(h, 1),
        be1.reshape(h, 1),
        w2,
        b2.reshape(h, 1),
        g2n.reshape(h, 1),
        be2.reshape(h, 1),
        w3,
        b3.reshape(1, 1),
    )
    return out


def kernel(x, edge_index, W1, b1, g1, be1, W2, b2, g2, be2, W3, b3):
    n, d = x.shape
    e = edge_index.shape[1]
    h = W2.shape[0]
    tab_a, tab_b = _make_tables(x, W1[:d], W1[d:], b1)
    gpaired = _gather_add(edge_index[0], edge_index[1], tab_a, tab_b)
    out = _edge_head(gpaired, g1, be1, W2, b2, g2, be2, W3, b3, block=_HEAD_BLOCK)
    return out.reshape(e)


# speedup vs baseline: 1.3466x; 1.3466x over previous
import functools

import jax
import jax.numpy as jnp
import numpy as np
from jax import lax
from jax.experimental import pallas as pl
from jax.experimental.pallas import tpu as pltpu
from jax.experimental.pallas import tpu_sc as plsc

_NUM_CORES = 2
_NUM_SUBCORES = 16
_NUM_WORKERS = _NUM_CORES * _NUM_SUBCORES
_CHUNK = 80
_HEAD_BLOCK = 1280


def _tables_body(x_ref, wa_ref, wb_ref, b1_ref, a_ref, b_ref):
    xv = x_ref[...]
    a_ref[...] = (
        jnp.dot(xv, wa_ref[...], preferred_element_type=jnp.float32) + b1_ref[...]
    )
    b_ref[...] = jnp.dot(xv, wb_ref[...], preferred_element_type=jnp.float32)


def _make_tables(x, w1a, w1b, b1):
    n, _ = x.shape
    h = w1a.shape[1]
    return pl.pallas_call(
        _tables_body,
        out_shape=[
            jax.ShapeDtypeStruct((n, h), jnp.float32),
            jax.ShapeDtypeStruct((n, h), jnp.float32),
        ],
    )(x, w1a, w1b, b1.reshape(1, h))


def _gather_add(start, end, tab_a, tab_b):
    e = start.shape[0]
    h = tab_a.shape[1]
    per_worker = e // _NUM_WORKERS
    chunks = per_worker // _CHUNK
    pairs = chunks // 2
    mesh = plsc.VectorSubcoreMesh(core_axis_name="c", subcore_axis_name="s")

    chunks_per_half = _HEAD_BLOCK // _CHUNK

    start2 = start.reshape(e // _CHUNK, _CHUNK)
    end2 = end.reshape(e // _CHUNK, _CHUNK)

    @functools.partial(
        pl.kernel,
        mesh=mesh,
        out_type=jax.ShapeDtypeStruct((e // 2, 2 * h), jnp.float32),
        compiler_params=pltpu.CompilerParams(use_tc_tiling_on_sc=False),
        scratch_types=[
            pltpu.VMEM((chunks, _CHUNK), jnp.int32),
            pltpu.VMEM((chunks, _CHUNK), jnp.int32),
            pltpu.VMEM((_CHUNK, h), jnp.float32),
            pltpu.VMEM((_CHUNK, h), jnp.float32),
            pltpu.VMEM((_CHUNK, h), jnp.float32),
            pltpu.VMEM((_CHUNK, h), jnp.float32),
            pltpu.VMEM((_CHUNK, h), jnp.float32),
            pltpu.VMEM((_CHUNK, h), jnp.float32),
            pltpu.SemaphoreType.DMA,
            pltpu.SemaphoreType.DMA,
            pltpu.SemaphoreType.DMA,
            pltpu.SemaphoreType.DMA,
        ],
    )
    def sc_kernel(start_hbm, end_hbm, a_hbm, b_hbm, g_hbm,
                  idx_s, idx_e, a0, b0, a1, b1, o0, o1, sg0, sg1, st0, st1):
        wid = lax.axis_index("s") * _NUM_CORES + lax.axis_index("c")
        cbase = wid * chunks

        def fire(c, ba, bb, sem):
            pltpu.async_copy(a_hbm.at[idx_s.at[c]], ba, sem)
            pltpu.async_copy(b_hbm.at[idx_e.at[c]], bb, sem)

        def wait_g(ba, bb, sem):
            pltpu.make_async_copy(a_hbm.at[idx_s.at[0]], ba, sem).wait()
            pltpu.make_async_copy(b_hbm.at[idx_e.at[0]], bb, sem).wait()

        def add_rows(ba, bb, oo):
            def row_body(p, c2):
                for t in range(h // 16):
                    sl = pl.ds(t * 16, 16)
                    oo[p, sl] = ba[p, sl] + bb[p, sl]
                return c2

            lax.fori_loop(0, _CHUNK, row_body, 0)

        def out_slice(c):
            gc = cbase + c
            blk = lax.shift_right_logical(gc, 5)
            half = lax.bitwise_and(lax.shift_right_logical(gc, 4), 1)
            j0 = lax.bitwise_and(gc, chunks_per_half - 1) * _CHUNK
            rowbase = pl.multiple_of(blk * _HEAD_BLOCK + j0, 8)
            colbase = pl.multiple_of(half * h, 8)
            return g_hbm.at[pl.ds(rowbase, _CHUNK), pl.ds(colbase, h)]

        def wait_st(oo, sem):
            pltpu.make_async_copy(oo, out_slice(0), sem).wait()

        pltpu.sync_copy(start_hbm.at[pl.ds(cbase, chunks)], idx_s)
        pltpu.sync_copy(end_hbm.at[pl.ds(cbase, chunks)], idx_e)
        fire(0, a0, b0, sg0)

        def pair_body(j, carry):
            c = 2 * j

            @pl.when(j > 0)
            def _():
                wait_st(o0, st0)
                wait_st(o1, st1)

            fire(c + 1, a1, b1, sg1)
            wait_g(a0, b0, sg0)
            add_rows(a0, b0, o0)
            pltpu.async_copy(o0, out_slice(c), st0)
            fire(c + 2, a0, b0, sg0)
            wait_g(a1, b1, sg1)
            add_rows(a1, b1, o1)
            pltpu.async_copy(o1, out_slice(c + 1), st1)
            return carry

        lax.fori_loop(0, pairs, pair_body, 0)

        wait_g(a0, b0, sg0)
        wait_st(o0, st0)
        add_rows(a0, b0, o0)
        pltpu.sync_copy(o0, out_slice(chunks - 1))
        wait_st(o1, st1)

    return sc_kernel(start2, end2, tab_a, tab_b)


def _ln_tanh_stack(s, gain, bias):
    mu = jnp.mean(s, axis=1, keepdims=True)
    d = s - jnp.broadcast_to(mu, s.shape)
    var = jnp.mean(d * d, axis=1, keepdims=True)
    r = jnp.broadcast_to(jax.lax.rsqrt(var + 1e-5), s.shape)
    return jnp.tanh(d * r * gain[None] + bias[None])


def _head_body(g_ref, g1_ref, be1_ref, w2_ref, b2_ref, g2_ref, be2_ref, w3_ref, b3_ref, out_ref):
    hh, block = g_ref.shape[1], g_ref.shape[0]
    h = hh // 2
    v = jnp.transpose(g_ref[...])
    s = _ln_tanh_stack(v.reshape(2, h, block), g1_ref[...], be1_ref[...])

    def half_dot(w, x):
        return lax.dot_general(
            w, x, (((0,), (0,)), ((), ())), preferred_element_type=jnp.float32
        )

    w2 = w2_ref[...]
    s = jnp.stack([half_dot(w2, s[0]), half_dot(w2, s[1])]) + b2_ref[...][None]
    s = _ln_tanh_stack(s, g2_ref[...], be2_ref[...])
    w3 = w3_ref[...]
    res = jnp.concatenate([half_dot(w3, s[0]), half_dot(w3, s[1])]) + b3_ref[...]
    out_ref[...] = res[None]


def _edge_head(g2, g1, be1, w2, b2, g2n, be2, w3, b3, block):
    rows, hh = g2.shape
    h = hh // 2
    grid = rows // block
    full = lambda i: (0, 0)

    out = pl.pallas_call(
        _head_body,
        grid=(grid,),
        in_specs=[
            pl.BlockSpec((block, hh), lambda i: (i, 0)),
            pl.BlockSpec((h, 1), full),
            pl.BlockSpec((h, 1), full),
            pl.BlockSpec((h, h), full),
            pl.BlockSpec((h, 1), full),
            pl.BlockSpec((h, 1), full),
            pl.BlockSpec((h, 1), full),
            pl.BlockSpec((h, 1), full),
            pl.BlockSpec((1, 1), full),
        ],
        out_specs=pl.BlockSpec((1, 2, block), lambda i: (i, 0, 0)),
        out_shape=jax.ShapeDtypeStruct((grid, 2, block), jnp.float32),
        compiler_params=pltpu.CompilerParams(
            dimension_semantics=("parallel",),
        ),
    )(
        g2,
        g1.reshape(h, 1),
        be1.reshape(h, 1),
        w2,
        b2.reshape(h, 1),
        g2n.reshape(h, 1),
        be2.reshape(h, 1),
        w3,
        b3.reshape(1, 1),
    )
    return out


def kernel(x, edge_index, W1, b1, g1, be1, W2, b2, g2, be2, W3, b3):
    n, d = x.shape
    e = edge_index.shape[1]
    h = W2.shape[0]
    tab_a, tab_b = _make_tables(x, W1[:d], W1[d:], b1)
    gpaired = _gather_add(edge_index[0], edge_index[1], tab_a, tab_b)
    out = _edge_head(gpaired, g1, be1, W2, b2, g2, be2, W3, b3, block=_HEAD_BLOCK)
    return out.reshape(e)


# speedup vs baseline: 1.7629x; 1.3091x over previous
import functools

import jax
import jax.numpy as jnp
import numpy as np
from jax import lax
from jax.experimental import pallas as pl
from jax.experimental.pallas import tpu as pltpu
from jax.experimental.pallas import tpu_sc as plsc

_NUM_CORES = 2
_NUM_SUBCORES = 16
_NUM_WORKERS = _NUM_CORES * _NUM_SUBCORES
_CHUNK = 80
_N_PARTS = 5
_PART_HEAD_BLOCK = 1600


def _tables_body(x_ref, wa_ref, wb_ref, b1_ref, a_ref, b_ref):
    xv = x_ref[...]
    a_ref[...] = (
        jnp.dot(xv, wa_ref[...], preferred_element_type=jnp.float32) + b1_ref[...]
    )
    b_ref[...] = jnp.dot(xv, wb_ref[...], preferred_element_type=jnp.float32)


def _make_tables(x, w1a, w1b, b1):
    n, _ = x.shape
    h = w1a.shape[1]
    return pl.pallas_call(
        _tables_body,
        out_shape=[
            jax.ShapeDtypeStruct((n, h), jnp.float32),
            jax.ShapeDtypeStruct((n, h), jnp.float32),
        ],
    )(x, w1a, w1b, b1.reshape(1, h))


def _gather_add(start2, end2, tab_a, tab_b, part, n_parts, head_block):
    e = start2.size // n_parts
    h = tab_a.shape[1]
    chunks = e // _NUM_WORKERS // _CHUNK
    pairs = chunks // 2
    part_cbase = part * (e // _CHUNK)
    mesh = plsc.VectorSubcoreMesh(core_axis_name="c", subcore_axis_name="s")

    chunks_per_half = head_block // _CHUNK
    chunks_per_block = 2 * chunks_per_half

    @functools.partial(
        pl.kernel,
        mesh=mesh,
        out_type=jax.ShapeDtypeStruct((e // 2, 2 * h), jnp.float32),
        compiler_params=pltpu.CompilerParams(use_tc_tiling_on_sc=False),
        scratch_types=[
            pltpu.VMEM((chunks, _CHUNK), jnp.int32),
            pltpu.VMEM((chunks, _CHUNK), jnp.int32),
            pltpu.VMEM((_CHUNK, h), jnp.float32),
            pltpu.VMEM((_CHUNK, h), jnp.float32),
            pltpu.VMEM((_CHUNK, h), jnp.float32),
            pltpu.VMEM((_CHUNK, h), jnp.float32),
            pltpu.VMEM((_CHUNK, h), jnp.float32),
            pltpu.VMEM((_CHUNK, h), jnp.float32),
            pltpu.SemaphoreType.DMA,
            pltpu.SemaphoreType.DMA,
            pltpu.SemaphoreType.DMA,
            pltpu.SemaphoreType.DMA,
        ],
    )
    def sc_kernel(start_hbm, end_hbm, a_hbm, b_hbm, g_hbm,
                  idx_s, idx_e, a0, b0, a1, b1, o0, o1, sg0, sg1, st0, st1):
        wid = lax.axis_index("s") * _NUM_CORES + lax.axis_index("c")
        cbase = part_cbase + wid * chunks

        def fire(c, ba, bb, sem):
            pltpu.async_copy(a_hbm.at[idx_s.at[c]], ba, sem)
            pltpu.async_copy(b_hbm.at[idx_e.at[c]], bb, sem)

        def wait_g(ba, bb, sem):
            pltpu.make_async_copy(a_hbm.at[idx_s.at[0]], ba, sem).wait()
            pltpu.make_async_copy(b_hbm.at[idx_e.at[0]], bb, sem).wait()

        def add_rows(ba, bb, oo):
            def row_body(p, c2):
                for t in range(h // 16):
                    sl = pl.ds(t * 16, 16)
                    oo[p, sl] = ba[p, sl] + bb[p, sl]
                return c2

            lax.fori_loop(0, _CHUNK, row_body, 0)

        def out_slice(c):
            gc = cbase - part_cbase + c
            blk = gc // chunks_per_block
            rem = gc % chunks_per_block
            half = rem // chunks_per_half
            j0 = (rem % chunks_per_half) * _CHUNK
            rowbase = pl.multiple_of(blk * head_block + j0, 8)
            colbase = pl.multiple_of(half * h, 8)
            return g_hbm.at[pl.ds(rowbase, _CHUNK), pl.ds(colbase, h)]

        def wait_st(oo, sem):
            pltpu.make_async_copy(oo, out_slice(0), sem).wait()

        pltpu.sync_copy(start_hbm.at[pl.ds(cbase, chunks)], idx_s)
        pltpu.sync_copy(end_hbm.at[pl.ds(cbase, chunks)], idx_e)
        fire(0, a0, b0, sg0)

        def pair_body(j, carry):
            c = 2 * j

            @pl.when(j > 0)
            def _():
                wait_st(o0, st0)
                wait_st(o1, st1)

            fire(c + 1, a1, b1, sg1)
            wait_g(a0, b0, sg0)
            add_rows(a0, b0, o0)
            pltpu.async_copy(o0, out_slice(c), st0)
            fire(c + 2, a0, b0, sg0)
            wait_g(a1, b1, sg1)
            add_rows(a1, b1, o1)
            pltpu.async_copy(o1, out_slice(c + 1), st1)
            return carry

        lax.fori_loop(0, pairs, pair_body, 0)

        wait_g(a0, b0, sg0)
        wait_st(o0, st0)
        add_rows(a0, b0, o0)
        pltpu.sync_copy(o0, out_slice(chunks - 1))
        wait_st(o1, st1)

    return sc_kernel(start2, end2, tab_a, tab_b)


def _ln_tanh_stack(s, gain, bias):
    mu = jnp.mean(s, axis=1, keepdims=True)
    d = s - jnp.broadcast_to(mu, s.shape)
    var = jnp.mean(d * d, axis=1, keepdims=True)
    r = jnp.broadcast_to(jax.lax.rsqrt(var + 1e-5), s.shape)
    return jnp.tanh(d * r * gain[None] + bias[None])


def _head_body(g_ref, g1_ref, be1_ref, w2_ref, b2_ref, g2_ref, be2_ref, w3_ref, b3_ref, out_ref):
    hh, block = g_ref.shape[1], g_ref.shape[0]
    h = hh // 2
    v = jnp.transpose(g_ref[...])
    s = _ln_tanh_stack(v.reshape(2, h, block), g1_ref[...], be1_ref[...])

    def half_dot(w, x):
        return lax.dot_general(
            w, x, (((0,), (0,)), ((), ())), preferred_element_type=jnp.float32
        )

    w2 = w2_ref[...]
    s = jnp.stack([half_dot(w2, s[0]), half_dot(w2, s[1])]) + b2_ref[...][None]
    s = _ln_tanh_stack(s, g2_ref[...], be2_ref[...])
    w3 = w3_ref[...]
    res = jnp.concatenate([half_dot(w3, s[0]), half_dot(w3, s[1])]) + b3_ref[...]
    out_ref[...] = res[None]


def _edge_head(g2, g1, be1, w2, b2, g2n, be2, w3, b3, block):
    rows, hh = g2.shape
    h = hh // 2
    grid = rows // block
    full = lambda i: (0, 0)

    out = pl.pallas_call(
        _head_body,
        grid=(grid,),
        in_specs=[
            pl.BlockSpec((block, hh), lambda i: (i, 0)),
            pl.BlockSpec((h, 1), full),
            pl.BlockSpec((h, 1), full),
            pl.BlockSpec((h, h), full),
            pl.BlockSpec((h, 1), full),
            pl.BlockSpec((h, 1), full),
            pl.BlockSpec((h, 1), full),
            pl.BlockSpec((h, 1), full),
            pl.BlockSpec((1, 1), full),
        ],
        out_specs=pl.BlockSpec((1, 2, block), lambda i: (i, 0, 0)),
        out_shape=jax.ShapeDtypeStruct((grid, 2, block), jnp.float32),
        compiler_params=pltpu.CompilerParams(
            dimension_semantics=("parallel",),
        ),
    )(
        g2,
        g1.reshape(h, 1),
        be1.reshape(h, 1),
        w2,
        b2.reshape(h, 1),
        g2n.reshape(h, 1),
        be2.reshape(h, 1),
        w3,
        b3.reshape(1, 1),
    )
    return out


def kernel(x, edge_index, W1, b1, g1, be1, W2, b2, g2, be2, W3, b3):
    n, d = x.shape
    e = edge_index.shape[1]
    tab_a, tab_b = _make_tables(x, W1[:d], W1[d:], b1)
    start2 = edge_index[0].reshape(e // _CHUNK, _CHUNK)
    end2 = edge_index[1].reshape(e // _CHUNK, _CHUNK)
    outs = []
    for p in range(_N_PARTS):
        gp = _gather_add(start2, end2, tab_a, tab_b, p, _N_PARTS, _PART_HEAD_BLOCK)
        outs.append(
            _edge_head(
                gp, g1, be1, W2, b2, g2, be2, W3, b3, block=_PART_HEAD_BLOCK
            ).reshape(-1)
        )
    return jnp.concatenate(outs)
